# baseline (speedup 1.0000x reference)
import functools

import jax
import jax.numpy as jnp
from jax import lax
from jax.experimental import pallas as pl
from jax.experimental.pallas import tpu as pltpu
from jax.experimental.pallas import tpu_sc as plsc

N_ROWS = 320000
N_SEG = 10000
HALF = N_SEG // 2
ACC_ROWS = 5040
D = 128
W = 128
N_WIN = N_ROWS // W
CHUNK = 312
TAIL = ACC_ROWS - 16 * CHUNK
OUT_TAIL = HALF - 16 * CHUNK
NBUF = 4
MAXSTEPS = (N_WIN + 15) // 16
N_GROUPS = (MAXSTEPS + NBUF - 1) // NBUF


def _sc_body(src_hbm, batch_hbm, zeros_hbm, rsplit_hbm, out_hbm,
             rbuf0, rbuf1, rbuf2, rbuf3, ids0, ids1, ids2, ids3,
             rvmem, acc, lsem0, lsem1, lsem2, lsem3):
    c = lax.axis_index("c")
    s = lax.axis_index("s")

    ids = [ids0, ids1, ids2, ids3]
    rbuf = [rbuf0, rbuf1, rbuf2, rbuf3]
    lsem = [lsem0, lsem1, lsem2, lsem3]

    pltpu.sync_copy(rsplit_hbm, rvmem)
    rstar = rvmem[pl.ds(0, 16)][0]
    lo = c * HALF
    wlo = jnp.where(c == 0, 0, rstar // W)
    whi = jnp.where(c == 0, (rstar + W - 1) // W, N_WIN)

    def start_load(k, b):
        win = wlo + s + k * 16

        @pl.when(win < whi)
        def _():
            pltpu.async_copy(batch_hbm.at[pl.ds(win * W, W)], ids[b], lsem[b])
            pltpu.async_copy(src_hbm.at[pl.ds(win * W, W)], rbuf[b], lsem[b])

    for b in range(NBUF - 1):
        start_load(b, b)

    pltpu.sync_copy(zeros_hbm.at[pl.ds(s * CHUNK, CHUNK)],
                    acc.at[pl.ds(s * CHUNK, CHUNK)])

    @pl.when(s == 15)
    def _():
        pltpu.sync_copy(zeros_hbm.at[pl.ds(16 * CHUNK, TAIL)],
                        acc.at[pl.ds(16 * CHUNK, TAIL)])

    plsc.subcore_barrier()

    def group(g, carry):
        base = g * NBUF
        for j in range(NBUF):
            k = base + j
            bc = j
            bl = (j + NBUF - 1) % NBUF
            win = wlo + s + k * 16

            start_load(k + NBUF - 1, bl)

            @pl.when(win < whi)
            def _(bc=bc):
                pltpu.make_async_copy(batch_hbm.at[pl.ds(0, W)], ids[bc],
                                      lsem[bc]).wait()
                pltpu.make_async_copy(src_hbm.at[pl.ds(0, W)], rbuf[bc],
                                      lsem[bc]).wait()
                for t in range(W // 16):
                    v = ids[bc][pl.ds(16 * t, 16)]
                    vl = v - lo
                    ok = (vl >= 0) & (vl < HALF)
                    ids[bc][pl.ds(16 * t, 16)] = jnp.where(ok, vl, HALF)
                pltpu.sync_copy(rbuf[bc], acc.at[ids[bc]], add=True)
        return carry

    lax.fori_loop(0, N_GROUPS, group, 0)

    plsc.subcore_barrier()
    pltpu.sync_copy(acc.at[pl.ds(s * CHUNK, CHUNK)],
                    out_hbm.at[pl.ds(c * HALF + s * CHUNK, CHUNK)])

    @pl.when(s == 15)
    def _():
        pltpu.sync_copy(acc.at[pl.ds(16 * CHUNK, OUT_TAIL)],
                        out_hbm.at[pl.ds(c * HALF + 16 * CHUNK, OUT_TAIL)])


@functools.partial(
    pl.kernel,
    out_type=jax.ShapeDtypeStruct((N_SEG, D), jnp.float32),
    mesh=plsc.VectorSubcoreMesh(core_axis_name="c", subcore_axis_name="s"),
    scratch_types=(
        [pltpu.VMEM((W, D), jnp.float32)] * NBUF
        + [pltpu.VMEM((W,), jnp.int32)] * NBUF
        + [pltpu.VMEM((16,), jnp.int32)]
        + [pltpu.VMEM_SHARED((ACC_ROWS, D), jnp.float32)]
        + [pltpu.SemaphoreType.DMA] * NBUF
    ),
)
def _sc_scatter_add(*refs):
    _sc_body(*refs)


def kernel(src, batch, dim_size):
    batch32 = jnp.asarray(batch, jnp.int32)
    rsplit = jnp.full((16,), jnp.searchsorted(batch32, jnp.int32(HALF)),
                      dtype=jnp.int32)
    zeros = jnp.zeros((ACC_ROWS, D), jnp.float32)
    return _sc_scatter_add(src, batch32, zeros, rsplit)


# speedup vs baseline: 1.2599x; 1.2599x over previous
import functools

import jax
import jax.numpy as jnp
from jax import lax
from jax.experimental import pallas as pl
from jax.experimental.pallas import tpu as pltpu
from jax.experimental.pallas import tpu_sc as plsc

N_ROWS = 320000
N_SEG = 10000
HALF = N_SEG // 2
ACC_ROWS = 5040
D = 128
W = 128
N_WIN = N_ROWS // W
CHUNK = 312
TAIL = ACC_ROWS - 16 * CHUNK
OUT_TAIL = HALF - 16 * CHUNK
NBUF = 4
MAXSTEPS = (N_WIN + 15) // 16
N_GROUPS = (MAXSTEPS + NBUF - 1) // NBUF


def _sc_body(src_hbm, batch_hbm, zeros_hbm, rsplit_hbm, out_hbm,
             rbuf0, rbuf1, rbuf2, rbuf3, ids0, ids1, ids2, ids3,
             rvmem, acc, lsem0, lsem1, lsem2, lsem3):
    c = lax.axis_index("c")
    s = lax.axis_index("s")

    ids = [ids0, ids1, ids2, ids3]
    rbuf = [rbuf0, rbuf1, rbuf2, rbuf3]
    lsem = [lsem0, lsem1, lsem2, lsem3]

    pltpu.sync_copy(rsplit_hbm, rvmem)
    rstar = rvmem[pl.ds(0, 16)][0]
    wlo = jnp.where(c == 0, 0, rstar // W)
    whi = jnp.where(c == 0, (rstar + W - 1) // W, N_WIN)

    def start_load(k, b):
        win = wlo + s + k * 16

        @pl.when(win < whi)
        def _():
            pltpu.async_copy(batch_hbm.at[pl.ds(c * N_ROWS + win * W, W)],
                             ids[b], lsem[b])
            pltpu.async_copy(src_hbm.at[pl.ds(win * W, W)], rbuf[b], lsem[b])

    for b in range(NBUF - 1):
        start_load(b, b)

    pltpu.sync_copy(zeros_hbm.at[pl.ds(s * CHUNK, CHUNK)],
                    acc.at[pl.ds(s * CHUNK, CHUNK)])

    @pl.when(s == 15)
    def _():
        pltpu.sync_copy(zeros_hbm.at[pl.ds(16 * CHUNK, TAIL)],
                        acc.at[pl.ds(16 * CHUNK, TAIL)])

    plsc.subcore_barrier()

    def group(g, carry):
        base = g * NBUF
        for j in range(NBUF):
            k = base + j
            bc = j
            bl = (j + NBUF - 1) % NBUF
            win = wlo + s + k * 16

            start_load(k + NBUF - 1, bl)

            @pl.when(win < whi)
            def _(bc=bc):
                pltpu.make_async_copy(batch_hbm.at[pl.ds(0, W)], ids[bc],
                                      lsem[bc]).wait()
                pltpu.make_async_copy(src_hbm.at[pl.ds(0, W)], rbuf[bc],
                                      lsem[bc]).wait()
                pltpu.sync_copy(rbuf[bc], acc.at[ids[bc]], add=True)
        return carry

    lax.fori_loop(0, N_GROUPS, group, 0)

    plsc.subcore_barrier()
    pltpu.sync_copy(acc.at[pl.ds(s * CHUNK, CHUNK)],
                    out_hbm.at[pl.ds(c * HALF + s * CHUNK, CHUNK)])

    @pl.when(s == 15)
    def _():
        pltpu.sync_copy(acc.at[pl.ds(16 * CHUNK, OUT_TAIL)],
                        out_hbm.at[pl.ds(c * HALF + 16 * CHUNK, OUT_TAIL)])


@functools.partial(
    pl.kernel,
    out_type=jax.ShapeDtypeStruct((N_SEG, D), jnp.float32),
    mesh=plsc.VectorSubcoreMesh(core_axis_name="c", subcore_axis_name="s"),
    scratch_types=(
        [pltpu.VMEM((W, D), jnp.float32)] * NBUF
        + [pltpu.VMEM((W,), jnp.int32)] * NBUF
        + [pltpu.VMEM((16,), jnp.int32)]
        + [pltpu.VMEM_SHARED((ACC_ROWS, D), jnp.float32)]
        + [pltpu.SemaphoreType.DMA] * NBUF
    ),
)
def _sc_scatter_add(*refs):
    _sc_body(*refs)


def kernel(src, batch, dim_size):
    batch32 = jnp.asarray(batch, jnp.int32)
    ids_sc0 = jnp.where(batch32 < HALF, batch32, HALF)
    ids_sc1 = jnp.where(batch32 >= HALF, batch32 - HALF, HALF)
    ids_cat = jnp.concatenate([ids_sc0, ids_sc1])
    rsplit = jnp.full((16,), jnp.sum(batch32 < HALF, dtype=jnp.int32))
    zeros = jnp.zeros((ACC_ROWS, D), jnp.float32)
    return _sc_scatter_add(src, ids_cat, zeros, rsplit)


# speedup vs baseline: 1.2754x; 1.0123x over previous
import functools

import jax
import jax.numpy as jnp
from jax import lax
from jax.experimental import pallas as pl
from jax.experimental.pallas import tpu as pltpu
from jax.experimental.pallas import tpu_sc as plsc

N_ROWS = 320000
N_SEG = 10000
HALF = N_SEG // 2
ACC_ROWS = 5040
D = 128
W = 128
N_WIN = N_ROWS // W
CHUNK = 312
TAIL = ACC_ROWS - 16 * CHUNK
OUT_TAIL = HALF - 16 * CHUNK
NBUF = 4
MAXSTEPS = (N_WIN + 15) // 16
N_GROUPS = (MAXSTEPS + 2 + NBUF - 1) // NBUF


def _sc_body(src_hbm, batch_hbm, zeros_hbm, rsplit_hbm, out_hbm,
             rbuf0, rbuf1, rbuf2, rbuf3, ids0, ids1, ids2, ids3,
             rvmem, acc, lsem0, lsem1, lsem2, lsem3,
             ssem0, ssem1, ssem2, ssem3):
    c = lax.axis_index("c")
    s = lax.axis_index("s")

    ids = [ids0, ids1, ids2, ids3]
    rbuf = [rbuf0, rbuf1, rbuf2, rbuf3]
    lsem = [lsem0, lsem1, lsem2, lsem3]
    ssem = [ssem0, ssem1, ssem2, ssem3]

    pltpu.sync_copy(rsplit_hbm, rvmem)
    rstar = rvmem[pl.ds(0, 16)][0]
    wlo = jnp.where(c == 0, 0, rstar // W)
    whi = jnp.where(c == 0, (rstar + W - 1) // W, N_WIN)

    def start_load(k, b):
        win = wlo + s + k * 16

        @pl.when(win < whi)
        def _():
            pltpu.async_copy(batch_hbm.at[pl.ds(c * N_ROWS + win * W, W)],
                             ids[b], lsem[b])
            pltpu.async_copy(src_hbm.at[pl.ds(win * W, W)], rbuf[b], lsem[b])

    for b in range(2):
        start_load(b, b)

    pltpu.sync_copy(zeros_hbm.at[pl.ds(s * CHUNK, CHUNK)],
                    acc.at[pl.ds(s * CHUNK, CHUNK)])

    @pl.when(s == 15)
    def _():
        pltpu.sync_copy(zeros_hbm.at[pl.ds(16 * CHUNK, TAIL)],
                        acc.at[pl.ds(16 * CHUNK, TAIL)])

    plsc.subcore_barrier()

    def group(g, carry):
        base = g * NBUF
        for j in range(NBUF):
            k = base + j
            bc = j
            bd = (j + 2) % NBUF
            win = wlo + s + k * 16
            win_d = wlo + s + (k - 2) * 16

            @pl.when((k >= 2) & (win_d < whi))
            def _(bd=bd):
                pltpu.make_async_copy(rbuf[bd], acc.at[ids[bd]],
                                      ssem[bd]).wait()

            start_load(k + 2, bd)

            @pl.when(win < whi)
            def _(bc=bc):
                pltpu.make_async_copy(batch_hbm.at[pl.ds(0, W)], ids[bc],
                                      lsem[bc]).wait()
                pltpu.make_async_copy(src_hbm.at[pl.ds(0, W)], rbuf[bc],
                                      lsem[bc]).wait()
                pltpu.async_copy(rbuf[bc], acc.at[ids[bc]], ssem[bc],
                                 add=True)
        return carry

    lax.fori_loop(0, N_GROUPS, group, 0)

    plsc.subcore_barrier()
    pltpu.sync_copy(acc.at[pl.ds(s * CHUNK, CHUNK)],
                    out_hbm.at[pl.ds(c * HALF + s * CHUNK, CHUNK)])

    @pl.when(s == 15)
    def _():
        pltpu.sync_copy(acc.at[pl.ds(16 * CHUNK, OUT_TAIL)],
                        out_hbm.at[pl.ds(c * HALF + 16 * CHUNK, OUT_TAIL)])


@functools.partial(
    pl.kernel,
    out_type=jax.ShapeDtypeStruct((N_SEG, D), jnp.float32),
    mesh=plsc.VectorSubcoreMesh(core_axis_name="c", subcore_axis_name="s"),
    scratch_types=(
        [pltpu.VMEM((W, D), jnp.float32)] * NBUF
        + [pltpu.VMEM((W,), jnp.int32)] * NBUF
        + [pltpu.VMEM((16,), jnp.int32)]
        + [pltpu.VMEM_SHARED((ACC_ROWS, D), jnp.float32)]
        + [pltpu.SemaphoreType.DMA] * (2 * NBUF)
    ),
)
def _sc_scatter_add(*refs):
    _sc_body(*refs)


def kernel(src, batch, dim_size):
    batch32 = jnp.asarray(batch, jnp.int32)
    ids_sc0 = jnp.where(batch32 < HALF, batch32, HALF)
    ids_sc1 = jnp.where(batch32 >= HALF, batch32 - HALF, HALF)
    ids_cat = jnp.concatenate([ids_sc0, ids_sc1])
    rsplit = jnp.full((16,), jnp.sum(batch32 < HALF, dtype=jnp.int32))
    zeros = jnp.zeros((ACC_ROWS, D), jnp.float32)
    return _sc_scatter_add(src, ids_cat, zeros, rsplit)
